# baseline (speedup 1.0000x reference)
import functools

import jax
import jax.numpy as jnp
from jax import lax
from jax.experimental import pallas as pl
from jax.experimental.pallas import tpu as pltpu
from jax.experimental.pallas import tpu_sc as plsc

EMBED = 32
BATCH = 16384

_info = plsc.get_sparse_core_info()
_NC, _NS = _info.num_cores, _info.num_subcores
_NW = _NC * _NS
_BPW = BATCH // _NW
_CHUNK = 128
_NCHUNK = _BPW // _CHUNK


def _sc_gather(user_table, item_table, uidx, iidx):
    mesh = plsc.VectorSubcoreMesh(core_axis_name="c", subcore_axis_name="s")

    @functools.partial(
        pl.kernel,
        mesh=mesh,
        compiler_params=pltpu.CompilerParams(use_tc_tiling_on_sc=False),
        out_type=[
            jax.ShapeDtypeStruct((BATCH, EMBED), jnp.float32),
            jax.ShapeDtypeStruct((BATCH, EMBED), jnp.float32),
        ],
        scratch_types=[
            pltpu.VMEM((_NCHUNK, _CHUNK), jnp.int32),
            pltpu.VMEM((_NCHUNK, _CHUNK), jnp.int32),
            pltpu.VMEM((_BPW, EMBED), jnp.float32),
            pltpu.VMEM((_BPW, EMBED), jnp.float32),
            pltpu.SemaphoreType.DMA,
        ],
    )
    def k(ut_hbm, it_hbm, ui_hbm, ii_hbm, ue_out, ie_out,
          ui_v, ii_v, ur_v, ir_v, sem):
        wid = lax.axis_index("s") * _NC + lax.axis_index("c")
        base = wid * _BPW
        pltpu.sync_copy(ui_hbm.at[wid], ui_v)
        pltpu.sync_copy(ii_hbm.at[wid], ii_v)
        cps = []
        for j in range(_NCHUNK):
            cps.append(pltpu.async_copy(
                ut_hbm.at[ui_v.at[j]],
                ur_v.at[pl.ds(j * _CHUNK, _CHUNK)], sem))
            cps.append(pltpu.async_copy(
                it_hbm.at[ii_v.at[j]],
                ir_v.at[pl.ds(j * _CHUNK, _CHUNK)], sem))
        for cp in cps:
            cp.wait()
        pltpu.sync_copy(ur_v, ue_out.at[pl.ds(base, _BPW)])
        pltpu.sync_copy(ir_v, ie_out.at[pl.ds(base, _BPW)])

    return k(user_table, item_table, uidx, iidx)


def _mlp_body(ue_ref, ie_ref, w1u_ref, w1i_ref, b1_ref, w2_ref, b2_ref,
              w3_ref, b3_ref, out_ref):
    u = ue_ref[...]
    v = ie_ref[...]
    h = (jnp.dot(u, w1u_ref[...], preferred_element_type=jnp.float32)
         + jnp.dot(v, w1i_ref[...], preferred_element_type=jnp.float32)
         + b1_ref[...])
    h = jnp.maximum(h, 0.0)
    h2 = jnp.dot(h, w2_ref[...], preferred_element_type=jnp.float32) + b2_ref[...]
    h2 = jnp.maximum(h2, 0.0)
    z = jnp.sum(h2 * w3_ref[...], axis=1) + b3_ref[0, 0]
    out_ref[...] = 1.0 / (1.0 + jnp.exp(-z))


def _tc_mlp(ue, ie, w1u, w1i, b1r, w2t, b2r, w3r, b3r):
    blk = 2048
    grid = (BATCH // blk,)
    full = lambda shape: pl.BlockSpec(shape, lambda i: (0,) * len(shape))
    return pl.pallas_call(
        _mlp_body,
        grid=grid,
        in_specs=[
            pl.BlockSpec((blk, EMBED), lambda i: (i, 0)),
            pl.BlockSpec((blk, EMBED), lambda i: (i, 0)),
            full((EMBED, 64)),
            full((EMBED, 64)),
            full((1, 64)),
            full((64, EMBED)),
            full((1, EMBED)),
            full((1, EMBED)),
            full((1, 1)),
        ],
        out_specs=pl.BlockSpec((blk,), lambda i: (i,)),
        out_shape=jax.ShapeDtypeStruct((BATCH,), jnp.float32),
    )(ue, ie, w1u, w1i, b1r, w2t, b2r, w3r, b3r)


def kernel(batch_data, user_table, item_table, W1, b1, W2, b2, W3, b3):
    uidx = batch_data[:, 0].reshape(_NW, _NCHUNK, _CHUNK)
    iidx = batch_data[:, 1].reshape(_NW, _NCHUNK, _CHUNK)
    ue, ie = _sc_gather(user_table, item_table, uidx, iidx)
    w1t = W1.T
    w1u = w1t[:EMBED]
    w1i = w1t[EMBED:]
    return _tc_mlp(ue, ie, w1u, w1i, b1.reshape(1, 64), W2.T,
                   b2.reshape(1, EMBED), W3, b3.reshape(1, 1))
